# baseline (speedup 1.0000x reference)
import functools

import jax
import jax.numpy as jnp
from jax import lax
from jax.experimental import pallas as pl
from jax.experimental.pallas import tpu as pltpu
from jax.experimental.pallas import tpu_sc as plsc

N_NODES = 100000
HIDDEN = 128
NUM_SEGMENTS = 1024
CHUNK = 128
NUM_FULL = N_NODES // CHUNK
TAIL = N_NODES - NUM_FULL * CHUNK
NUM_WORKERS = 32
K_MAX = (NUM_FULL + NUM_WORKERS - 1) // NUM_WORKERS
ROWS_PER_SUBCORE = NUM_SEGMENTS // 16


def _sc_pool(x_hbm, ids_hbm, zeros_hbm, acc_hbm, ids_v, tail_ids_v, xbuf, shared_acc):
    cid = lax.axis_index("c")
    sid = lax.axis_index("s")
    wid = sid * 2 + cid

    pltpu.sync_copy(zeros_hbm, shared_acc.at[pl.ds(sid * ROWS_PER_SUBCORE,
                                                   ROWS_PER_SUBCORE)])
    plsc.subcore_barrier()

    @pl.loop(0, K_MAX)
    def _(k):
        chunk = wid + NUM_WORKERS * k

        @pl.when(chunk < NUM_FULL)
        def _():
            base = chunk * CHUNK
            pltpu.sync_copy(ids_hbm.at[pl.ds(base, CHUNK)], ids_v.at[0])
            pltpu.sync_copy(x_hbm.at[pl.ds(base, CHUNK)], xbuf)
            pltpu.sync_copy(xbuf, shared_acc.at[ids_v.at[0]], add=True)

    @pl.when(wid == NUM_WORKERS - 1)
    def _():
        base = NUM_FULL * CHUNK
        pltpu.sync_copy(ids_hbm.at[pl.ds(base, TAIL)], tail_ids_v.at[0])
        pltpu.sync_copy(x_hbm.at[pl.ds(base, TAIL)], xbuf.at[pl.ds(0, TAIL)])
        pltpu.sync_copy(xbuf.at[pl.ds(0, TAIL)],
                        shared_acc.at[tail_ids_v.at[0]], add=True)

    plsc.subcore_barrier()

    sl = pl.ds(sid * ROWS_PER_SUBCORE, ROWS_PER_SUBCORE)
    pltpu.sync_copy(shared_acc.at[sl], acc_hbm.at[cid].at[sl])


def _combine(acc_ref, out_ref):
    out_ref[...] = acc_ref[0] + acc_ref[1]


def kernel(node_states, segment_ids):
    ids32 = segment_ids.astype(jnp.int32)
    zeros = jnp.zeros((ROWS_PER_SUBCORE, HIDDEN), jnp.float32)

    sc_pool = pl.kernel(
        _sc_pool,
        out_type=jax.ShapeDtypeStruct((2, NUM_SEGMENTS, HIDDEN), jnp.float32),
        mesh=plsc.VectorSubcoreMesh(core_axis_name="c", subcore_axis_name="s"),
        scratch_types=[
            pltpu.VMEM((1, CHUNK), jnp.int32),
            pltpu.VMEM((1, TAIL), jnp.int32),
            pltpu.VMEM((CHUNK, HIDDEN), jnp.float32),
            pltpu.VMEM_SHARED((NUM_SEGMENTS, HIDDEN), jnp.float32),
        ],
    )
    acc = sc_pool(node_states, ids32, zeros)

    return pl.pallas_call(
        _combine,
        out_shape=jax.ShapeDtypeStruct((NUM_SEGMENTS, HIDDEN), jnp.float32),
    )(acc)


# speedup vs baseline: 1.4973x; 1.4973x over previous
import functools

import jax
import jax.numpy as jnp
from jax import lax
from jax.experimental import pallas as pl
from jax.experimental import pallas as pl_
from jax.experimental.pallas import tpu as pltpu
from jax.experimental.pallas import tpu_sc as plsc

N_NODES = 100000
HIDDEN = 128
NUM_SEGMENTS = 1024
CHUNK = 128
NUM_WORKERS = 32
K_UNIF = 24
NUM_UNIF = K_UNIF * NUM_WORKERS
NUM_FULL = N_NODES // CHUNK
NUM_EXTRA = NUM_FULL - NUM_UNIF
TAIL = N_NODES - NUM_FULL * CHUNK
NBUF = 4
ROWS_PER_SUBCORE = NUM_SEGMENTS // 16


def _sc_pool(x_hbm, ids2d_hbm, ids1d_hbm, zeros_hbm, acc_hbm,
             ids_all, extra_ids_v, tail_ids_v, xbuf, shared_acc,
             load_sems, scat_sems):
    cid = lax.axis_index("c")
    sid = lax.axis_index("s")
    wid = sid * 2 + cid

    pltpu.sync_copy(zeros_hbm, shared_acc.at[pl.ds(sid * ROWS_PER_SUBCORE,
                                                   ROWS_PER_SUBCORE)])
    plsc.subcore_barrier()

    start = wid * K_UNIF
    pltpu.sync_copy(ids2d_hbm.at[pl.ds(start, K_UNIF)], ids_all)

    def load(k, b):
        return pltpu.async_copy(
            x_hbm.at[pl.ds((start + k) * CHUNK, CHUNK)], xbuf.at[b],
            load_sems.at[b])

    lh = {k: load(k, k % NBUF) for k in range(NBUF)}
    sh = {}
    for k in range(K_UNIF):
        b = k % NBUF
        lh[k].wait()
        sh[k] = pltpu.async_copy(xbuf.at[b], shared_acc.at[ids_all.at[k]],
                                 scat_sems.at[b], add=True)
        if k + NBUF < K_UNIF:
            sh[k].wait()
            lh[k + NBUF] = load(k + NBUF, b)
    for k in range(K_UNIF - NBUF, K_UNIF):
        sh[k].wait()

    @pl.when(wid < NUM_EXTRA)
    def _():
        base = (NUM_UNIF + wid) * CHUNK
        pltpu.sync_copy(ids2d_hbm.at[pl.ds(NUM_UNIF + wid, 1)], extra_ids_v)
        pltpu.sync_copy(x_hbm.at[pl.ds(base, CHUNK)], xbuf.at[0])
        pltpu.sync_copy(xbuf.at[0], shared_acc.at[extra_ids_v.at[0]], add=True)

    @pl.when(wid == NUM_WORKERS - 1)
    def _():
        base = NUM_FULL * CHUNK
        pltpu.sync_copy(ids1d_hbm.at[pl.ds(base, TAIL)], tail_ids_v.at[0])
        pltpu.sync_copy(x_hbm.at[pl.ds(base, TAIL)], xbuf.at[0].at[pl.ds(0, TAIL)])
        pltpu.sync_copy(xbuf.at[0].at[pl.ds(0, TAIL)],
                        shared_acc.at[tail_ids_v.at[0]], add=True)

    plsc.subcore_barrier()

    sl = pl.ds(sid * ROWS_PER_SUBCORE, ROWS_PER_SUBCORE)
    pltpu.sync_copy(shared_acc.at[sl], acc_hbm.at[cid].at[sl])


def _combine(acc_ref, out_ref):
    out_ref[...] = acc_ref[0] + acc_ref[1]


def kernel(node_states, segment_ids):
    ids32 = segment_ids.astype(jnp.int32)
    ids2d = ids32[:NUM_FULL * CHUNK].reshape(NUM_FULL, CHUNK)
    zeros = jnp.zeros((ROWS_PER_SUBCORE, HIDDEN), jnp.float32)

    sc_pool = pl.kernel(
        _sc_pool,
        out_type=jax.ShapeDtypeStruct((2, NUM_SEGMENTS, HIDDEN), jnp.float32),
        mesh=plsc.VectorSubcoreMesh(core_axis_name="c", subcore_axis_name="s"),
        scratch_types=[
            pltpu.VMEM((K_UNIF, CHUNK), jnp.int32),
            pltpu.VMEM((1, CHUNK), jnp.int32),
            pltpu.VMEM((1, TAIL), jnp.int32),
            pltpu.VMEM((NBUF, CHUNK, HIDDEN), jnp.float32),
            pltpu.VMEM_SHARED((NUM_SEGMENTS, HIDDEN), jnp.float32),
            pltpu.SemaphoreType.DMA((NBUF,)),
            pltpu.SemaphoreType.DMA((NBUF,)),
        ],
    )
    acc = sc_pool(node_states, ids2d, ids32, zeros)

    return pl.pallas_call(
        _combine,
        out_shape=jax.ShapeDtypeStruct((NUM_SEGMENTS, HIDDEN), jnp.float32),
    )(acc)
